# baseline (speedup 1.0000x reference)
import functools
import jax
import jax.numpy as jnp
import numpy as np
from jax import lax
from jax.experimental import pallas as pl
from jax.experimental.pallas import tpu as pltpu
from jax.experimental.pallas import tpu_sc as plsc

ALPHA = 0.2
EPS = 1e-5
NOUT = 200
NFILT = 32
DT = 200
TYP = 1000
NE = 160000
NE_PAD = 163840
EPT = 5120
NT = 10240
DUMP = 4096
CR = 4352
RPT = 136
DCOL = 256
NR_PAD = 512
PEND = 160
BATCH = 4096

_mesh = plsc.VectorSubcoreMesh(core_axis_name="c", subcore_axis_name="s")
_CP = pltpu.CompilerParams(needs_layout_passes=False)


def _h_kernel(e_ref, w_ref, a_ref, h_ref, ha1_ref, ha2_ref):
    h = jnp.dot(e_ref[...], w_ref[...], preferred_element_type=jnp.float32)
    h_ref[:, :NOUT] = h
    h_ref[:, NOUT:] = jnp.zeros((e_ref.shape[0], DCOL - NOUT), jnp.float32)
    hs = jnp.dot(h, a_ref[...], preferred_element_type=jnp.float32)
    ha1_ref[...] = hs[:, 0].reshape(1, -1)
    ha2_ref[...] = hs[:, 1].reshape(1, -1)


def _h_matmul(E_pad, W, acols):
    BLK = 1024
    grid = (NT // BLK,)
    return pl.pallas_call(
        _h_kernel,
        grid=grid,
        in_specs=[
            pl.BlockSpec((BLK, 128), lambda i: (i, 0)),
            pl.BlockSpec((128, NOUT), lambda i: (0, 0)),
            pl.BlockSpec((NOUT, 2), lambda i: (0, 0)),
        ],
        out_specs=[
            pl.BlockSpec((BLK, DCOL), lambda i: (i, 0)),
            pl.BlockSpec((1, BLK), lambda i: (0, i)),
            pl.BlockSpec((1, BLK), lambda i: (0, i)),
        ],
        out_shape=[
            jax.ShapeDtypeStruct((NT, DCOL), jnp.float32),
            jax.ShapeDtypeStruct((1, NT), jnp.float32),
            jax.ShapeDtypeStruct((1, NT), jnp.float32),
        ],
    )(E_pad, W, acols)


def _r_kernel(r_ref, w_ref, a_ref, rp_ref, ra2_ref):
    r = jnp.dot(r_ref[...], w_ref[...], preferred_element_type=jnp.float32)
    rp_ref[:, :NOUT] = r
    rp_ref[:, NOUT:] = jnp.zeros((NR_PAD, DCOL - NOUT), jnp.float32)
    ra2_ref[...] = jnp.dot(r, a_ref[...], preferred_element_type=jnp.float32).reshape(1, -1)


def _r_matmul(R_pad, Wr, a2):
    return pl.pallas_call(
        _r_kernel,
        out_shape=[
            jax.ShapeDtypeStruct((NR_PAD, DCOL), jnp.float32),
            jax.ShapeDtypeStruct((1, NR_PAD), jnp.float32),
        ],
    )(R_pad, Wr, a2)


def _sred_kernel(sp_ref, s_ref):
    s_ref[...] = jnp.sum(sp_ref[...], axis=0, keepdims=True)


def _s_reduce(spart):
    return pl.pallas_call(
        _sred_kernel,
        out_shape=jax.ShapeDtypeStruct((1, CR), jnp.float32),
    )(spart)


@functools.partial(
    pl.kernel,
    out_type=[
        jax.ShapeDtypeStruct((32, 20480), jnp.int32),
        jax.ShapeDtypeStruct((32, 16), jnp.int32),
        jax.ShapeDtypeStruct((32, CR), jnp.float32),
        jax.ShapeDtypeStruct((NT,), jnp.int32),
    ],
    mesh=_mesh,
    scratch_types=[
        pltpu.VMEM((NT,), jnp.int32),
        pltpu.VMEM((NT,), jnp.float32),
        pltpu.VMEM((NT,), jnp.float32),
        pltpu.VMEM((NR_PAD,), jnp.float32),
        pltpu.VMEM((CR,), jnp.float32),
        pltpu.VMEM((EPT,), jnp.int32),
        pltpu.VMEM((EPT,), jnp.int32),
        pltpu.VMEM((EPT,), jnp.int32),
        pltpu.VMEM((20480,), jnp.int32),
        pltpu.VMEM((BATCH,), jnp.int32),
        pltpu.VMEM((16,), jnp.int32),
        pltpu.VMEM_SHARED((NT,), jnp.int32),
    ],
    compiler_params=_CP,
)
def _phase1(src_hbm, dst_hbm, et_hbm, xb_hbm, ha1_hbm, ha2_hbm, ra2_hbm,
            cdat, cnts, spart, remap_out,
            remap_v, ha1_v, ha2_v, ra2_v, s_v, src_v, dst_v, et_v,
            odat, xb_v, cnt16, remap_sh):
    cid = lax.axis_index("c")
    sid = lax.axis_index("s")
    w = cid * 16 + sid

    zero16i = jnp.zeros((16,), jnp.int32)
    zero16f = jnp.zeros((16,), jnp.float32)
    ones16 = jnp.full((16,), 1, jnp.int32)

    @pl.when(sid == 0)
    def _():
        def z(i, _):
            remap_v[pl.ds(i * 16, 16)] = zero16i
            return 0
        lax.fori_loop(0, NT // 16, z, 0)
        pltpu.sync_copy(xb_hbm, xb_v)

        def mark(i, _):
            plsc.store_scatter(remap_v, [xb_v[pl.ds(i * 16, 16)]], ones16)
            return 0
        lax.fori_loop(0, BATCH // 16, mark, 0)

        def slots(i, c):
            m16 = remap_v[pl.ds(i * 16, 16)]
            inc = plsc.cumsum(m16)
            slot = c + inc - 1
            remap_v[pl.ds(i * 16, 16)] = jnp.where(m16 == 1, slot,
                                                   jnp.full((16,), DUMP, jnp.int32))
            return c + jnp.max(inc)
        lax.fori_loop(0, NT // 16, slots, jnp.int32(0))
        pltpu.sync_copy(remap_v, remap_sh)

    plsc.subcore_barrier()

    @pl.when(sid != 0)
    def _():
        pltpu.sync_copy(remap_sh, remap_v)

    @pl.when(w == 0)
    def _():
        pltpu.sync_copy(remap_v, remap_out)

    pltpu.sync_copy(ha1_hbm, ha1_v)
    pltpu.sync_copy(ha2_hbm, ha2_v)
    pltpu.sync_copy(ra2_hbm, ra2_v)
    base = w * EPT
    pltpu.sync_copy(src_hbm.at[pl.ds(base, EPT)], src_v)
    pltpu.sync_copy(dst_hbm.at[pl.ds(base, EPT)], dst_v)
    pltpu.sync_copy(et_hbm.at[pl.ds(base, EPT)], et_v)

    def zs(i, _):
        s_v[pl.ds(i * 16, 16)] = zero16f
        return 0
    lax.fori_loop(0, CR // 16, zs, 0)

    dump16 = jnp.full((16,), DUMP, jnp.int32)

    def zo(i, _):
        fld = (i >> 5) & 3
        odat[pl.ds(i * 16, 16)] = jnp.where(fld == 2, dump16, zero16i)
        return 0
    lax.fori_loop(0, 20480 // 16, zo, 0)

    def body(i, cnt):
        o = i * 16
        s16 = src_v[pl.ds(o, 16)]
        d16 = dst_v[pl.ds(o, 16)]
        e16 = et_v[pl.ds(o, 16)]
        g = (plsc.load_gather(ha1_v, [d16]) + plsc.load_gather(ha2_v, [s16])
             - plsc.load_gather(ra2_v, [e16]))
        lk = jnp.maximum(g, ALPHA * g)
        ex = jnp.exp(lk)
        rd = plsc.load_gather(remap_v, [d16])
        plsc.addupdate_scatter(s_v, [rd], ex)
        msk = rd < DUMP
        inc = plsc.cumsum(jnp.where(msk, 1, 0))
        pos = cnt + inc - 1
        fbase = ((pos >> 9) << 11) + (pos & 511)
        plsc.store_scatter(odat, [fbase], s16, mask=msk)
        plsc.store_scatter(odat, [fbase + 512], e16, mask=msk)
        plsc.store_scatter(odat, [fbase + 1024], rd, mask=msk)
        plsc.store_scatter(odat, [fbase + 1536], plsc.bitcast(ex, jnp.int32),
                           mask=msk)
        return cnt + jnp.max(inc)

    cnt = lax.fori_loop(0, EPT // 16, body, jnp.int32(0))

    cnt16[...] = jnp.full((16,), cnt, jnp.int32)
    pltpu.sync_copy(cnt16, cnts.at[w])
    pltpu.sync_copy(odat, cdat.at[w])
    pltpu.sync_copy(s_v, spart.at[w])


CH2 = 512

@functools.partial(
    pl.kernel,
    out_type=jax.ShapeDtypeStruct((CR, DCOL), jnp.float32),
    mesh=_mesh,
    scratch_types=[
        pltpu.VMEM((RPT, DCOL), jnp.float32),
        pltpu.VMEM((128, DCOL), jnp.float32),
        pltpu.VMEM((128, DCOL), jnp.float32),
        pltpu.VMEM((2, 2048), jnp.int32),
        pltpu.VMEM((PEND,), jnp.int32),
        pltpu.VMEM((PEND,), jnp.int32),
        pltpu.VMEM((PEND,), jnp.int32),
        pltpu.VMEM((PEND,), jnp.float32),
        pltpu.VMEM((512,), jnp.int32),
        pltpu.SemaphoreType.DMA,
        pltpu.SemaphoreType.DMA,
        pltpu.SemaphoreType.DMA,
    ],
    compiler_params=_CP,
)
def _phase2(cdat, cnts_flat, h_hbm, r_hbm, accout,
            acc_v, bufH, bufR, dat_c,
            psrc, pet, prd, pex, cnts_v, semh, semr, semc):
    cid = lax.axis_index("c")
    sid = lax.axis_index("s")
    w = cid * 16 + sid
    lo = w * RPT

    zero16f = jnp.zeros((16,), jnp.float32)
    zero16i = jnp.zeros((16,), jnp.int32)

    def zacc(i, _):
        for j in range(DCOL // 16):
            acc_v[i, pl.ds(j * 16, 16)] = zero16f
        return 0
    lax.fori_loop(0, RPT, zacc, 0)
    for q in range(PEND // 16):
        psrc[pl.ds(q * 16, 16)] = zero16i
        pet[pl.ds(q * 16, 16)] = zero16i
        prd[pl.ds(q * 16, 16)] = jnp.full((16,), lo, jnp.int32)
        pex[pl.ds(q * 16, 16)] = zero16f
    pltpu.sync_copy(cnts_flat, cnts_v)

    def nch_of(p):
        cv = cnts_v[pl.ds(p * 16, 16)]
        return jnp.maximum((cv[0] + CH2 - 1) // CH2, 1)

    def issue(p, c, par):
        pltpu.async_copy(cdat.at[p, pl.ds(c * 2048, 2048)], dat_c.at[par], semc)

    def drain(par):
        pltpu.make_async_copy(cdat.at[0, pl.ds(0, 2048)], dat_c.at[par],
                              semc).wait()

    def fire(npend):
        ch = pltpu.async_copy(h_hbm.at[psrc.at[pl.ds(0, 128)]], bufH, semh)
        cr = pltpu.async_copy(r_hbm.at[pet.at[pl.ds(0, 128)]], bufR, semr)
        ch.wait()
        cr.wait()

        def grp(g, _):
            o = g * 16
            rd16 = prd[pl.ds(o, 16)] - lo
            ex16 = pex[pl.ds(o, 16)]
            for l in range(16):
                r = rd16[l]
                ex = ex16[l]
                for j in range(NOUT // 16 + 1):
                    c = (bufH[o + l, pl.ds(j * 16, 16)]
                         - bufR[o + l, pl.ds(j * 16, 16)]) * ex
                    plsc.addupdate(acc_v.at[r, pl.ds(j * 16, 16)], c)
            return 0
        lax.fori_loop(0, 8, grp, 0)
        t_src = psrc[pl.ds(128, 16)]
        t_et = pet[pl.ds(128, 16)]
        t_rd = prd[pl.ds(128, 16)]
        t_ex = pex[pl.ds(128, 16)]
        psrc[pl.ds(0, 16)] = t_src
        pet[pl.ds(0, 16)] = t_et
        prd[pl.ds(0, 16)] = t_rd
        pex[pl.ds(0, 16)] = t_ex
        return npend - 128

    def scan_group(g, carry):
        npend, par = carry
        o = g * 16
        rd16 = dat_c[par, pl.ds(1024 + o, 16)]
        ex16 = plsc.bitcast(dat_c[par, pl.ds(1536 + o, 16)], jnp.float32)
        own = jnp.logical_and(jnp.logical_and(rd16 >= lo, rd16 < lo + RPT),
                              ex16 > 0.0)
        pop = plsc.all_reduce_population_count(own)[0]

        def do(np_):
            inc = plsc.cumsum(jnp.where(own, 1, 0))
            pos = np_ + inc - 1
            plsc.store_scatter(psrc, [pos], dat_c[par, pl.ds(o, 16)], mask=own)
            plsc.store_scatter(pet, [pos], dat_c[par, pl.ds(512 + o, 16)], mask=own)
            plsc.store_scatter(prd, [pos], rd16, mask=own)
            plsc.store_scatter(pex, [pos], ex16, mask=own)
            np2 = np_ + pop
            return lax.cond(np2 >= 128, fire, lambda n: n, np2)

        npend = lax.cond(pop > 0, do, lambda n: n, npend)
        return (npend, par)

    def chunk(p, c, nch, carry):
        npend, gc = carry
        par = gc % 2
        drain(par)
        @pl.when(c + 1 < nch)
        def _():
            issue(p, c + 1, (gc + 1) % 2)

        @pl.when(jnp.logical_and(c + 1 >= nch, p + 1 < 32))
        def _():
            issue(p + 1, 0, (gc + 1) % 2)

        npend, _unused = lax.fori_loop(0, CH2 // 16, scan_group, (npend, par))
        return (npend, gc + 1)

    def producer(p, carry):
        nch = nch_of(p)
        return lax.fori_loop(0, nch, lambda c, cr: chunk(p, c, nch, cr), carry)

    issue(0, 0, 0)
    npend, _gc = lax.fori_loop(0, 32, producer, (jnp.int32(0), jnp.int32(0)))

    @pl.when(npend > 0)
    def _():
        def pad(q, _):
            o = q * 16
            lanes = lax.iota(jnp.int32, 16) + o
            dead = lanes >= npend
            plsc.store_scatter(pex, [lax.iota(jnp.int32, 16) + o],
                               zero16f, mask=dead)
            plsc.store_scatter(prd, [lax.iota(jnp.int32, 16) + o],
                               jnp.full((16,), lo, jnp.int32), mask=dead)
            return 0
        lax.fori_loop(0, 8, pad, 0)
        fire(npend)

    pltpu.sync_copy(acc_v, accout.at[pl.ds(lo, RPT)])


@functools.partial(
    pl.kernel,
    out_type=jax.ShapeDtypeStruct((BATCH, DCOL), jnp.float32),
    mesh=_mesh,
    scratch_types=[
        pltpu.VMEM((NT,), jnp.int32),
        pltpu.VMEM((CR,), jnp.float32),
        pltpu.VMEM((128, DCOL), jnp.float32),
        pltpu.VMEM((128,), jnp.int32),
        pltpu.VMEM((128,), jnp.int32),
        pltpu.SemaphoreType.DMA,
    ],
    compiler_params=_CP,
)
def _phase3(accout, s_hbm, remap_hbm, xb_hbm, e_out,
            remap_v, s_v, bufA, xb_v, rd_v, sem):
    cid = lax.axis_index("c")
    sid = lax.axis_index("s")
    w = cid * 16 + sid
    base = w * 128

    pltpu.sync_copy(remap_hbm, remap_v)
    pltpu.sync_copy(s_hbm, s_v)
    pltpu.sync_copy(xb_hbm.at[pl.ds(base, 128)], xb_v)

    def mapg(g, _):
        o = g * 16
        rd_v[pl.ds(o, 16)] = plsc.load_gather(remap_v, [xb_v[pl.ds(o, 16)]])
        return 0
    lax.fori_loop(0, 8, mapg, 0)

    pltpu.async_copy(accout.at[rd_v], bufA, sem).wait()

    def rowg(g, _):
        o = g * 16
        s16 = plsc.load_gather(s_v, [rd_v[pl.ds(o, 16)]])
        inv16 = 1.0 / (s16 + 1e-16)
        for l in range(16):
            inv = inv16[l]
            for j in range(NOUT // 16 + 1):
                c = bufA[o + l, pl.ds(j * 16, 16)] * inv
                bufA[o + l, pl.ds(j * 16, 16)] = jnp.where(
                    c > 0.0, c, jnp.exp(c) - 1.0)
        return 0
    lax.fori_loop(0, 8, rowg, 0)

    pltpu.sync_copy(bufA, e_out.at[pl.ds(base, 128)])


def _decoder_kernel(x0_ref, x1_ref, x2_ref, x3_ref, A_ref, B_ref, D_ref,
                    fcw_ref, c2s_ref, c2b_ref, tt_ref, bout_ref, out_ref):
    acc = jnp.zeros((x0_ref.shape[0], DT), jnp.float32)
    x0 = x0_ref[...]
    x1 = x1_ref[...]
    x2 = x2_ref[...]
    x3 = x3_ref[...]
    for f in range(NFILT):
        a = A_ref[0, f]
        b = B_ref[0, f]
        d = D_ref[0, f]
        c1 = jnp.maximum(x0 * a + x1 * b + d, 0.0)
        c2 = jnp.maximum(x2 * a + x3 * b + d, 0.0)
        P = jnp.maximum(c1, c2)
        acc = acc + jnp.dot(P, fcw_ref[f], preferred_element_type=jnp.float32)
    y = jnp.maximum(acc * c2s_ref[...] + c2b_ref[...], 0.0)
    z = jnp.dot(y, tt_ref[...], preferred_element_type=jnp.float32) + bout_ref[...]
    out_ref[...] = jax.nn.sigmoid(z)


def _decoder(e, conv_w, conv_b, fc_w, fc_b, b_out, bn1_g, bn1_b,
             bn2_g, bn2_b, bn3_g, bn3_b, T):
    B = e.shape[0]
    s1 = bn1_g[0] / jnp.sqrt(1.0 + EPS)
    b1 = bn1_b[0]
    bn3s = bn3_g / jnp.sqrt(1.0 + EPS)
    cw0 = conv_w[:, 0, 0, 0]
    cw1 = conv_w[:, 0, 0, 1]
    A = (bn3s * cw0 * s1).reshape(1, NFILT)
    Bc = (bn3s * cw1 * s1).reshape(1, NFILT)
    D = (bn3s * ((cw0 + cw1) * b1 + conv_b) + bn3_b).reshape(1, NFILT)
    bn2s = bn2_g / jnp.sqrt(1.0 + EPS)
    c2s = bn2s.reshape(1, DT)
    c2b = (fc_b * bn2s + bn2_b).reshape(1, DT)
    fcw = fc_w.reshape(DT, NFILT, 50).transpose(1, 2, 0)
    tt = T.T
    x0 = e[:, 0::4]
    x1 = e[:, 1::4]
    x2 = e[:, 2::4]
    x3 = e[:, 3::4]
    BB = 512
    grid = (B // BB,)
    return pl.pallas_call(
        _decoder_kernel,
        grid=grid,
        in_specs=[
            pl.BlockSpec((BB, 50), lambda i: (i, 0)),
            pl.BlockSpec((BB, 50), lambda i: (i, 0)),
            pl.BlockSpec((BB, 50), lambda i: (i, 0)),
            pl.BlockSpec((BB, 50), lambda i: (i, 0)),
            pl.BlockSpec((1, NFILT), lambda i: (0, 0)),
            pl.BlockSpec((1, NFILT), lambda i: (0, 0)),
            pl.BlockSpec((1, NFILT), lambda i: (0, 0)),
            pl.BlockSpec((NFILT, 50, DT), lambda i: (0, 0, 0)),
            pl.BlockSpec((1, DT), lambda i: (0, 0)),
            pl.BlockSpec((1, DT), lambda i: (0, 0)),
            pl.BlockSpec((DT, TYP), lambda i: (0, 0)),
            pl.BlockSpec((1, TYP), lambda i: (0, 0)),
        ],
        out_specs=pl.BlockSpec((BB, TYP), lambda i: (i, 0)),
        out_shape=jax.ShapeDtypeStruct((B, TYP), jnp.float32),
    )(x0, x1, x2, x3, A, Bc, D, fcw, c2s, c2b, tt, b_out.reshape(1, TYP))


def kernel(x_batch, edge_index, edge_type, E, R, T, W_att, Wr_att, a_att,
           conv_w, conv_b, fc_w, fc_b, b_out, bn1_g, bn1_b, bn2_g, bn2_b,
           bn3_g, bn3_b):
    x_batch = x_batch.astype(jnp.int32)
    src = jnp.concatenate([edge_index[0].astype(jnp.int32),
                           jnp.zeros((NE_PAD - NE,), jnp.int32)])
    dst = jnp.concatenate([edge_index[1].astype(jnp.int32),
                           jnp.full((NE_PAD - NE,), 10000, jnp.int32)])
    et = jnp.concatenate([edge_type.astype(jnp.int32),
                          jnp.zeros((NE_PAD - NE,), jnp.int32)])

    a1 = a_att[:NOUT]
    a2 = a_att[NOUT:]
    acols = jnp.stack([a1, a2], axis=1)
    E_pad = jnp.concatenate([E, jnp.zeros((NT - E.shape[0], 128), jnp.float32)])
    R_pad = jnp.concatenate([R, jnp.zeros((NR_PAD - R.shape[0], 128), jnp.float32)])

    h_pad, ha1, ha2 = _h_matmul(E_pad, W_att, acols)
    r_pad, ra2 = _r_matmul(R_pad, Wr_att, a2.reshape(-1, 1))

    cdat, cnts, spart, remap = _phase1(
        src, dst, et, x_batch, ha1.reshape(NT), ha2.reshape(NT),
        ra2.reshape(NR_PAD))

    s_final = _s_reduce(spart).reshape(CR)

    accout = _phase2(cdat, cnts.reshape(512), h_pad, r_pad)

    e_pad = _phase3(accout, s_final, remap, x_batch)

    e = e_pad[:, :NOUT]
    return _decoder(e, conv_w, conv_b, fc_w, fc_b, b_out, bn1_g, bn1_b,
                    bn2_g, bn2_b, bn3_g, bn3_b, T)


# speedup vs baseline: 1.0391x; 1.0391x over previous
import functools
import jax
import jax.numpy as jnp
import numpy as np
from jax import lax
from jax.experimental import pallas as pl
from jax.experimental.pallas import tpu as pltpu
from jax.experimental.pallas import tpu_sc as plsc

ALPHA = 0.2
EPS = 1e-5
NOUT = 200
NFILT = 32
DT = 200
TYP = 1000
NE = 160000
NE_PAD = 163840
EPT = 5120
NT = 10240
DUMP = 4096
CR = 4352
RPT = 136
DCOL = 256
NR_PAD = 512
PEND = 160
BATCH = 4096

_mesh = plsc.VectorSubcoreMesh(core_axis_name="c", subcore_axis_name="s")
_CP = pltpu.CompilerParams(needs_layout_passes=False)


def _h_kernel(e_ref, w_ref, a_ref, h_ref, ha1_ref, ha2_ref):
    h = jnp.dot(e_ref[...], w_ref[...], preferred_element_type=jnp.float32)
    h_ref[:, :NOUT] = h
    h_ref[:, NOUT:] = jnp.zeros((e_ref.shape[0], DCOL - NOUT), jnp.float32)
    hs = jnp.dot(h, a_ref[...], preferred_element_type=jnp.float32)
    ha1_ref[...] = hs[:, 0].reshape(1, -1)
    ha2_ref[...] = hs[:, 1].reshape(1, -1)


def _h_matmul(E_pad, W, acols):
    BLK = 1024
    grid = (NT // BLK,)
    return pl.pallas_call(
        _h_kernel,
        grid=grid,
        in_specs=[
            pl.BlockSpec((BLK, 128), lambda i: (i, 0)),
            pl.BlockSpec((128, NOUT), lambda i: (0, 0)),
            pl.BlockSpec((NOUT, 2), lambda i: (0, 0)),
        ],
        out_specs=[
            pl.BlockSpec((BLK, DCOL), lambda i: (i, 0)),
            pl.BlockSpec((1, BLK), lambda i: (0, i)),
            pl.BlockSpec((1, BLK), lambda i: (0, i)),
        ],
        out_shape=[
            jax.ShapeDtypeStruct((NT, DCOL), jnp.float32),
            jax.ShapeDtypeStruct((1, NT), jnp.float32),
            jax.ShapeDtypeStruct((1, NT), jnp.float32),
        ],
    )(E_pad, W, acols)


def _r_kernel(r_ref, w_ref, a_ref, rp_ref, ra2_ref):
    r = jnp.dot(r_ref[...], w_ref[...], preferred_element_type=jnp.float32)
    rp_ref[:, :NOUT] = r
    rp_ref[:, NOUT:] = jnp.zeros((NR_PAD, DCOL - NOUT), jnp.float32)
    ra2_ref[...] = jnp.dot(r, a_ref[...], preferred_element_type=jnp.float32).reshape(1, -1)


def _r_matmul(R_pad, Wr, a2):
    return pl.pallas_call(
        _r_kernel,
        out_shape=[
            jax.ShapeDtypeStruct((NR_PAD, DCOL), jnp.float32),
            jax.ShapeDtypeStruct((1, NR_PAD), jnp.float32),
        ],
    )(R_pad, Wr, a2)


def _sred_kernel(sp_ref, s_ref):
    s_ref[...] = jnp.sum(sp_ref[...], axis=0, keepdims=True)


def _s_reduce(spart):
    return pl.pallas_call(
        _sred_kernel,
        out_shape=jax.ShapeDtypeStruct((1, CR), jnp.float32),
    )(spart)


@functools.partial(
    pl.kernel,
    out_type=[
        jax.ShapeDtypeStruct((32, 20480), jnp.int32),
        jax.ShapeDtypeStruct((32, 16), jnp.int32),
        jax.ShapeDtypeStruct((32, CR), jnp.float32),
        jax.ShapeDtypeStruct((NT,), jnp.int32),
    ],
    mesh=_mesh,
    scratch_types=[
        pltpu.VMEM((NT,), jnp.int32),
        pltpu.VMEM((NT,), jnp.float32),
        pltpu.VMEM((NT,), jnp.float32),
        pltpu.VMEM((NR_PAD,), jnp.float32),
        pltpu.VMEM((CR,), jnp.float32),
        pltpu.VMEM((EPT,), jnp.int32),
        pltpu.VMEM((EPT,), jnp.int32),
        pltpu.VMEM((EPT,), jnp.int32),
        pltpu.VMEM((20480,), jnp.int32),
        pltpu.VMEM((BATCH,), jnp.int32),
        pltpu.VMEM((16,), jnp.int32),
        pltpu.VMEM_SHARED((NT,), jnp.int32),
    ],
    compiler_params=_CP,
)
def _phase1(src_hbm, dst_hbm, et_hbm, xb_hbm, ha1_hbm, ha2_hbm, ra2_hbm,
            cdat, cnts, spart, remap_out,
            remap_v, ha1_v, ha2_v, ra2_v, s_v, src_v, dst_v, et_v,
            odat, xb_v, cnt16, remap_sh):
    cid = lax.axis_index("c")
    sid = lax.axis_index("s")
    w = cid * 16 + sid

    zero16i = jnp.zeros((16,), jnp.int32)
    zero16f = jnp.zeros((16,), jnp.float32)
    ones16 = jnp.full((16,), 1, jnp.int32)

    @pl.when(sid == 0)
    def _():
        def z(i, _):
            remap_v[pl.ds(i * 16, 16)] = zero16i
            return 0
        lax.fori_loop(0, NT // 16, z, 0)
        pltpu.sync_copy(xb_hbm, xb_v)

        def mark(i, _):
            plsc.store_scatter(remap_v, [xb_v[pl.ds(i * 16, 16)]], ones16)
            return 0
        lax.fori_loop(0, BATCH // 16, mark, 0)

        def slots(i, c):
            m16 = remap_v[pl.ds(i * 16, 16)]
            inc = plsc.cumsum(m16)
            slot = c + inc - 1
            remap_v[pl.ds(i * 16, 16)] = jnp.where(m16 == 1, slot,
                                                   jnp.full((16,), DUMP, jnp.int32))
            return c + jnp.max(inc)
        lax.fori_loop(0, NT // 16, slots, jnp.int32(0))
        pltpu.sync_copy(remap_v, remap_sh)

    plsc.subcore_barrier()

    @pl.when(sid != 0)
    def _():
        pltpu.sync_copy(remap_sh, remap_v)

    @pl.when(w == 0)
    def _():
        pltpu.sync_copy(remap_v, remap_out)

    pltpu.sync_copy(ha1_hbm, ha1_v)
    pltpu.sync_copy(ha2_hbm, ha2_v)
    pltpu.sync_copy(ra2_hbm, ra2_v)
    base = w * EPT
    pltpu.sync_copy(src_hbm.at[pl.ds(base, EPT)], src_v)
    pltpu.sync_copy(dst_hbm.at[pl.ds(base, EPT)], dst_v)
    pltpu.sync_copy(et_hbm.at[pl.ds(base, EPT)], et_v)

    def zs(i, _):
        s_v[pl.ds(i * 16, 16)] = zero16f
        return 0
    lax.fori_loop(0, CR // 16, zs, 0)

    dump16 = jnp.full((16,), DUMP, jnp.int32)

    def zo(i, _):
        fld = (i >> 5) & 3
        odat[pl.ds(i * 16, 16)] = jnp.where(fld == 2, dump16, zero16i)
        return 0
    lax.fori_loop(0, 20480 // 16, zo, 0)

    def body(i, cnt):
        o = i * 16
        s16 = src_v[pl.ds(o, 16)]
        d16 = dst_v[pl.ds(o, 16)]
        e16 = et_v[pl.ds(o, 16)]
        g = (plsc.load_gather(ha1_v, [d16]) + plsc.load_gather(ha2_v, [s16])
             - plsc.load_gather(ra2_v, [e16]))
        lk = jnp.maximum(g, ALPHA * g)
        ex = jnp.exp(lk)
        rd = plsc.load_gather(remap_v, [d16])
        plsc.addupdate_scatter(s_v, [rd], ex)
        msk = rd < DUMP
        inc = plsc.cumsum(jnp.where(msk, 1, 0))
        pos = cnt + inc - 1
        fbase = ((pos >> 9) << 11) + (pos & 511)
        plsc.store_scatter(odat, [fbase], s16, mask=msk)
        plsc.store_scatter(odat, [fbase + 512], e16, mask=msk)
        plsc.store_scatter(odat, [fbase + 1024], rd, mask=msk)
        plsc.store_scatter(odat, [fbase + 1536], plsc.bitcast(ex, jnp.int32),
                           mask=msk)
        return cnt + jnp.max(inc)

    cnt = lax.fori_loop(0, EPT // 16, body, jnp.int32(0))

    cnt16[...] = jnp.full((16,), cnt, jnp.int32)
    pltpu.sync_copy(cnt16, cnts.at[w])
    pltpu.sync_copy(odat, cdat.at[w])
    pltpu.sync_copy(s_v, spart.at[w])


CH2 = 512

@functools.partial(
    pl.kernel,
    out_type=jax.ShapeDtypeStruct((CR, DCOL), jnp.float32),
    mesh=_mesh,
    scratch_types=[
        pltpu.VMEM((RPT, DCOL), jnp.float32),
        pltpu.VMEM((128, DCOL), jnp.float32),
        pltpu.VMEM((128, DCOL), jnp.float32),
        pltpu.VMEM((2, 2048), jnp.int32),
        pltpu.VMEM((PEND,), jnp.int32),
        pltpu.VMEM((PEND,), jnp.int32),
        pltpu.VMEM((PEND,), jnp.int32),
        pltpu.VMEM((PEND,), jnp.float32),
        pltpu.VMEM((512,), jnp.int32),
        pltpu.SemaphoreType.DMA,
        pltpu.SemaphoreType.DMA,
        pltpu.SemaphoreType.DMA,
    ],
    compiler_params=_CP,
)
def _phase2(cdat, cnts_flat, h_hbm, r_hbm, accout,
            acc_v, bufH, bufR, dat_c,
            psrc, pet, prd, pex, cnts_v, semh, semr, semc):
    cid = lax.axis_index("c")
    sid = lax.axis_index("s")
    w = cid * 16 + sid
    lo = w * RPT

    zero16f = jnp.zeros((16,), jnp.float32)
    zero16i = jnp.zeros((16,), jnp.int32)

    def zacc(i, _):
        for j in range(DCOL // 16):
            acc_v[i, pl.ds(j * 16, 16)] = zero16f
        return 0
    lax.fori_loop(0, RPT, zacc, 0)
    for q in range(PEND // 16):
        psrc[pl.ds(q * 16, 16)] = zero16i
        pet[pl.ds(q * 16, 16)] = zero16i
        prd[pl.ds(q * 16, 16)] = jnp.full((16,), lo, jnp.int32)
        pex[pl.ds(q * 16, 16)] = zero16f
    pltpu.sync_copy(cnts_flat, cnts_v)

    def nch_of(p):
        cv = cnts_v[pl.ds(p * 16, 16)]
        return jnp.maximum((cv[0] + CH2 - 1) // CH2, 1)

    def issue(p, c, par):
        pltpu.async_copy(cdat.at[p, pl.ds(c * 2048, 2048)], dat_c.at[par], semc)

    def drain(par):
        pltpu.make_async_copy(cdat.at[0, pl.ds(0, 2048)], dat_c.at[par],
                              semc).wait()

    def fire(npend):
        ch = pltpu.async_copy(h_hbm.at[psrc.at[pl.ds(0, 128)]], bufH, semh)
        cr = pltpu.async_copy(r_hbm.at[pet.at[pl.ds(0, 128)]], bufR, semr)
        ch.wait()
        cr.wait()

        def grp(g, _):
            o = g * 16
            rd16 = prd[pl.ds(o, 16)] - lo
            ex16 = pex[pl.ds(o, 16)]
            for l in range(16):
                r = rd16[l]
                ex = ex16[l]
                for j in range(NOUT // 16 + 1):
                    c = (bufH[o + l, pl.ds(j * 16, 16)]
                         - bufR[o + l, pl.ds(j * 16, 16)]) * ex
                    plsc.addupdate(acc_v.at[r, pl.ds(j * 16, 16)], c)
            return 0
        lax.fori_loop(0, 8, grp, 0)
        t_src = psrc[pl.ds(128, 16)]
        t_et = pet[pl.ds(128, 16)]
        t_rd = prd[pl.ds(128, 16)]
        t_ex = pex[pl.ds(128, 16)]
        psrc[pl.ds(0, 16)] = t_src
        pet[pl.ds(0, 16)] = t_et
        prd[pl.ds(0, 16)] = t_rd
        pex[pl.ds(0, 16)] = t_ex
        return npend - 128

    def scan_group(g, carry):
        npend, par = carry
        o = g * 16
        rd16 = dat_c[par, pl.ds(1024 + o, 16)]
        ex16 = plsc.bitcast(dat_c[par, pl.ds(1536 + o, 16)], jnp.float32)
        own = jnp.logical_and(jnp.logical_and(rd16 >= lo, rd16 < lo + RPT),
                              ex16 > 0.0)
        inc = plsc.cumsum(jnp.where(own, 1, 0))
        pos = npend + inc - 1
        plsc.store_scatter(psrc, [pos], dat_c[par, pl.ds(o, 16)], mask=own)
        plsc.store_scatter(pet, [pos], dat_c[par, pl.ds(512 + o, 16)], mask=own)
        plsc.store_scatter(prd, [pos], rd16, mask=own)
        plsc.store_scatter(pex, [pos], ex16, mask=own)
        npend = npend + inc[15]
        npend = lax.cond(npend >= 128, fire, lambda n: n, npend)
        return (npend, par)

    def chunk(p, c, nch, carry):
        npend, gc = carry
        par = gc % 2
        drain(par)
        @pl.when(c + 1 < nch)
        def _():
            issue(p, c + 1, (gc + 1) % 2)

        @pl.when(jnp.logical_and(c + 1 >= nch, p + 1 < 32))
        def _():
            issue(p + 1, 0, (gc + 1) % 2)

        npend, _unused = lax.fori_loop(0, CH2 // 16, scan_group, (npend, par))
        return (npend, gc + 1)

    def producer(p, carry):
        nch = nch_of(p)
        return lax.fori_loop(0, nch, lambda c, cr: chunk(p, c, nch, cr), carry)

    issue(0, 0, 0)
    npend, _gc = lax.fori_loop(0, 32, producer, (jnp.int32(0), jnp.int32(0)))

    @pl.when(npend > 0)
    def _():
        def pad(q, _):
            o = q * 16
            lanes = lax.iota(jnp.int32, 16) + o
            dead = lanes >= npend
            plsc.store_scatter(pex, [lax.iota(jnp.int32, 16) + o],
                               zero16f, mask=dead)
            plsc.store_scatter(prd, [lax.iota(jnp.int32, 16) + o],
                               jnp.full((16,), lo, jnp.int32), mask=dead)
            return 0
        lax.fori_loop(0, 8, pad, 0)
        fire(npend)

    pltpu.sync_copy(acc_v, accout.at[pl.ds(lo, RPT)])


@functools.partial(
    pl.kernel,
    out_type=jax.ShapeDtypeStruct((BATCH, DCOL), jnp.float32),
    mesh=_mesh,
    scratch_types=[
        pltpu.VMEM((NT,), jnp.int32),
        pltpu.VMEM((CR,), jnp.float32),
        pltpu.VMEM((128, DCOL), jnp.float32),
        pltpu.VMEM((128,), jnp.int32),
        pltpu.VMEM((128,), jnp.int32),
        pltpu.SemaphoreType.DMA,
    ],
    compiler_params=_CP,
)
def _phase3(accout, s_hbm, remap_hbm, xb_hbm, e_out,
            remap_v, s_v, bufA, xb_v, rd_v, sem):
    cid = lax.axis_index("c")
    sid = lax.axis_index("s")
    w = cid * 16 + sid
    base = w * 128

    pltpu.sync_copy(remap_hbm, remap_v)
    pltpu.sync_copy(s_hbm, s_v)
    pltpu.sync_copy(xb_hbm.at[pl.ds(base, 128)], xb_v)

    def mapg(g, _):
        o = g * 16
        rd_v[pl.ds(o, 16)] = plsc.load_gather(remap_v, [xb_v[pl.ds(o, 16)]])
        return 0
    lax.fori_loop(0, 8, mapg, 0)

    pltpu.async_copy(accout.at[rd_v], bufA, sem).wait()

    def rowg(g, _):
        o = g * 16
        s16 = plsc.load_gather(s_v, [rd_v[pl.ds(o, 16)]])
        inv16 = 1.0 / (s16 + 1e-16)
        for l in range(16):
            inv = inv16[l]
            for j in range(NOUT // 16 + 1):
                c = bufA[o + l, pl.ds(j * 16, 16)] * inv
                bufA[o + l, pl.ds(j * 16, 16)] = jnp.where(
                    c > 0.0, c, jnp.exp(c) - 1.0)
        return 0
    lax.fori_loop(0, 8, rowg, 0)

    pltpu.sync_copy(bufA, e_out.at[pl.ds(base, 128)])


def _decoder_kernel(x0_ref, x1_ref, x2_ref, x3_ref, A_ref, B_ref, D_ref,
                    fcw_ref, c2s_ref, c2b_ref, tt_ref, bout_ref, out_ref):
    acc = jnp.zeros((x0_ref.shape[0], DT), jnp.float32)
    x0 = x0_ref[...]
    x1 = x1_ref[...]
    x2 = x2_ref[...]
    x3 = x3_ref[...]
    for f in range(NFILT):
        a = A_ref[0, f]
        b = B_ref[0, f]
        d = D_ref[0, f]
        c1 = jnp.maximum(x0 * a + x1 * b + d, 0.0)
        c2 = jnp.maximum(x2 * a + x3 * b + d, 0.0)
        P = jnp.maximum(c1, c2)
        acc = acc + jnp.dot(P, fcw_ref[f], preferred_element_type=jnp.float32)
    y = jnp.maximum(acc * c2s_ref[...] + c2b_ref[...], 0.0)
    z = jnp.dot(y, tt_ref[...], preferred_element_type=jnp.float32) + bout_ref[...]
    out_ref[...] = jax.nn.sigmoid(z)


def _decoder(e, conv_w, conv_b, fc_w, fc_b, b_out, bn1_g, bn1_b,
             bn2_g, bn2_b, bn3_g, bn3_b, T):
    B = e.shape[0]
    s1 = bn1_g[0] / jnp.sqrt(1.0 + EPS)
    b1 = bn1_b[0]
    bn3s = bn3_g / jnp.sqrt(1.0 + EPS)
    cw0 = conv_w[:, 0, 0, 0]
    cw1 = conv_w[:, 0, 0, 1]
    A = (bn3s * cw0 * s1).reshape(1, NFILT)
    Bc = (bn3s * cw1 * s1).reshape(1, NFILT)
    D = (bn3s * ((cw0 + cw1) * b1 + conv_b) + bn3_b).reshape(1, NFILT)
    bn2s = bn2_g / jnp.sqrt(1.0 + EPS)
    c2s = bn2s.reshape(1, DT)
    c2b = (fc_b * bn2s + bn2_b).reshape(1, DT)
    fcw = fc_w.reshape(DT, NFILT, 50).transpose(1, 2, 0)
    tt = T.T
    x0 = e[:, 0::4]
    x1 = e[:, 1::4]
    x2 = e[:, 2::4]
    x3 = e[:, 3::4]
    BB = 512
    grid = (B // BB,)
    return pl.pallas_call(
        _decoder_kernel,
        grid=grid,
        in_specs=[
            pl.BlockSpec((BB, 50), lambda i: (i, 0)),
            pl.BlockSpec((BB, 50), lambda i: (i, 0)),
            pl.BlockSpec((BB, 50), lambda i: (i, 0)),
            pl.BlockSpec((BB, 50), lambda i: (i, 0)),
            pl.BlockSpec((1, NFILT), lambda i: (0, 0)),
            pl.BlockSpec((1, NFILT), lambda i: (0, 0)),
            pl.BlockSpec((1, NFILT), lambda i: (0, 0)),
            pl.BlockSpec((NFILT, 50, DT), lambda i: (0, 0, 0)),
            pl.BlockSpec((1, DT), lambda i: (0, 0)),
            pl.BlockSpec((1, DT), lambda i: (0, 0)),
            pl.BlockSpec((DT, TYP), lambda i: (0, 0)),
            pl.BlockSpec((1, TYP), lambda i: (0, 0)),
        ],
        out_specs=pl.BlockSpec((BB, TYP), lambda i: (i, 0)),
        out_shape=jax.ShapeDtypeStruct((B, TYP), jnp.float32),
    )(x0, x1, x2, x3, A, Bc, D, fcw, c2s, c2b, tt, b_out.reshape(1, TYP))


def kernel(x_batch, edge_index, edge_type, E, R, T, W_att, Wr_att, a_att,
           conv_w, conv_b, fc_w, fc_b, b_out, bn1_g, bn1_b, bn2_g, bn2_b,
           bn3_g, bn3_b):
    x_batch = x_batch.astype(jnp.int32)
    src = jnp.concatenate([edge_index[0].astype(jnp.int32),
                           jnp.zeros((NE_PAD - NE,), jnp.int32)])
    dst = jnp.concatenate([edge_index[1].astype(jnp.int32),
                           jnp.full((NE_PAD - NE,), 10000, jnp.int32)])
    et = jnp.concatenate([edge_type.astype(jnp.int32),
                          jnp.zeros((NE_PAD - NE,), jnp.int32)])

    a1 = a_att[:NOUT]
    a2 = a_att[NOUT:]
    acols = jnp.stack([a1, a2], axis=1)
    E_pad = jnp.concatenate([E, jnp.zeros((NT - E.shape[0], 128), jnp.float32)])
    R_pad = jnp.concatenate([R, jnp.zeros((NR_PAD - R.shape[0], 128), jnp.float32)])

    h_pad, ha1, ha2 = _h_matmul(E_pad, W_att, acols)
    r_pad, ra2 = _r_matmul(R_pad, Wr_att, a2.reshape(-1, 1))

    cdat, cnts, spart, remap = _phase1(
        src, dst, et, x_batch, ha1.reshape(NT), ha2.reshape(NT),
        ra2.reshape(NR_PAD))

    s_final = _s_reduce(spart).reshape(CR)

    accout = _phase2(cdat, cnts.reshape(512), h_pad, r_pad)

    e_pad = _phase3(accout, s_final, remap, x_batch)

    e = e_pad[:, :NOUT]
    return _decoder(e, conv_w, conv_b, fc_w, fc_b, b_out, bn1_g, bn1_b,
                    bn2_g, bn2_b, bn3_g, bn3_b, T)
